# baseline (speedup 1.0000x reference)
import functools

import jax
import jax.numpy as jnp
from jax import lax
from jax.experimental import pallas as pl
from jax.experimental.pallas import tpu as pltpu
from jax.experimental.pallas import tpu_sc as plsc

_B = 4096
_L = 200
_D = 64
_VP = 500000
_NC = 2
_NS = 16
_NW = _NC * _NS
_BPW = _B // _NW
_IPW = _BPW * _L
_RU = 8
_R = 3


@functools.partial(
    pl.kernel,
    mesh=plsc.VectorSubcoreMesh(core_axis_name="c", subcore_axis_name="s"),
    out_type=jax.ShapeDtypeStruct((_B, _D), jnp.float32),
    scratch_types=[
        pltpu.VMEM((_IPW + 16,), jnp.int32),
        pltpu.VMEM((_R * _L,), jnp.int32),
        pltpu.VMEM((_R, _L, 2 * _D), jnp.float32),
        pltpu.VMEM((_BPW, _D), jnp.float32),
        pltpu.SemaphoreType.DMA,
        pltpu.SemaphoreType.DMA,
        pltpu.SemaphoreType.DMA,
    ],
)
def _glove_bow_sc(x_hbm, tab_hbm, out_hbm, idx_v, pidx_v, rows_v, out_v,
                  sem0, sem1, sem2):
    wid = lax.axis_index("s") * _NC + lax.axis_index("c")
    base = wid * _BPW
    pltpu.sync_copy(x_hbm.at[pl.ds(base * _L, _IPW)], idx_v.at[pl.ds(0, _IPW)])

    sems = (sem0, sem1, sem2)

    def copies(slot):
        return (
            pltpu.make_async_copy(
                tab_hbm.at[pidx_v.at[pl.ds(slot * _L, _L)]],
                rows_v.at[slot],
                sems[slot],
            ),
        )

    def issue(b, slot):
        for c in range(13):
            o = min(c * 16, _L - 16)
            vi = idx_v[pl.ds(b * _L + o, 16)]
            pidx_v[pl.ds(slot * _L + o, 16)] = jax.lax.shift_right_logical(vi, 1)
        for cp in copies(slot):
            cp.start()

    def wait(slot):
        for cp in copies(slot):
            cp.wait()

    def reduce(b, slot):
        zero = jnp.zeros((16,), jnp.float32)

        def rbody(i, acc):
            l0 = i * _RU
            vi = idx_v[pl.ds(b * _L + l0, 16)]
            offv = (vi & 1) * _D
            for u in range(_RU):
                l = l0 + u
                off = offv[u]
                acc = tuple(
                    acc[j] + rows_v[slot, l, pl.ds(off + j * 16, 16)]
                    for j in range(4)
                )
            return acc

        acc = lax.fori_loop(0, _L // _RU, rbody, (zero,) * 4)
        for j in range(4):
            out_v[b, pl.ds(j * 16, 16)] = acc[j]

    def step(b, slot):
        nb = b + (_R - 1)

        @pl.when(nb < _BPW)
        def _():
            issue(nb, (slot + _R - 1) % _R)

        wait(slot)
        reduce(b, slot)

    for r in range(_R - 1):
        issue(r, r)

    def outer(o, carry):
        b0 = o * _R
        for k in range(_R):
            step(b0 + k, k)
        return carry

    lax.fori_loop(0, _BPW // _R, outer, 0)

    for k in range(_BPW - _BPW % _R, _BPW):
        step(k, k % _R)

    pltpu.sync_copy(out_v, out_hbm.at[pl.ds(base, _BPW)])


def kernel(x, embed_weight):
    x1 = x.astype(jnp.int32).reshape(-1)
    wp = embed_weight.reshape(_VP, 2 * _D)
    return _glove_bow_sc(x1, wp)


# speedup vs baseline: 1.1332x; 1.1332x over previous
import functools

import jax
import jax.numpy as jnp
from jax import lax
from jax.experimental import pallas as pl
from jax.experimental.pallas import tpu as pltpu
from jax.experimental.pallas import tpu_sc as plsc

_B = 4096
_L = 200
_D = 64
_V = 1000000
_NC = 2
_NS = 16
_NW = _NC * _NS
_BPW = _B // _NW
_IPW = _BPW * _L
_RU = 8
_R = 3


@functools.partial(
    pl.kernel,
    mesh=plsc.VectorSubcoreMesh(core_axis_name="c", subcore_axis_name="s"),
    compiler_params=pltpu.CompilerParams(use_tc_tiling_on_sc=False),
    out_type=jax.ShapeDtypeStruct((_B, _D), jnp.float32),
    scratch_types=[
        pltpu.VMEM((_IPW,), jnp.int32),
        pltpu.VMEM((_R, _L, _D), jnp.float32),
        pltpu.VMEM((_BPW, _D), jnp.float32),
        pltpu.SemaphoreType.DMA,
        pltpu.SemaphoreType.DMA,
        pltpu.SemaphoreType.DMA,
    ],
)
def _glove_bow_sc(x_hbm, tab_hbm, out_hbm, idx_v, rows_v, out_v,
                  sem0, sem1, sem2):
    wid = lax.axis_index("s") * _NC + lax.axis_index("c")
    base = wid * _BPW
    pltpu.sync_copy(x_hbm.at[pl.ds(base * _L, _IPW)], idx_v)

    sems = (sem0, sem1, sem2)

    def copies(b, slot):
        return (
            pltpu.make_async_copy(
                tab_hbm.at[idx_v.at[pl.ds(b * _L, _L)]],
                rows_v.at[slot],
                sems[slot],
            ),
        )

    def issue(b, slot):
        for cp in copies(b, slot):
            cp.start()

    def wait(b, slot):
        for cp in copies(b, slot):
            cp.wait()

    def reduce(b, slot):
        zero = jnp.zeros((16,), jnp.float32)

        def rbody(i, acc):
            l0 = i * _RU
            for u in range(_RU):
                l = l0 + u
                acc = tuple(
                    acc[j] + rows_v[slot, l, pl.ds(j * 16, 16)]
                    for j in range(4)
                )
            return acc

        acc = lax.fori_loop(0, _L // _RU, rbody, (zero,) * 4)
        for j in range(4):
            out_v[b, pl.ds(j * 16, 16)] = acc[j]

    def step(b, slot):
        nb = b + (_R - 1)

        @pl.when(nb < _BPW)
        def _():
            issue(nb, (slot + _R - 1) % _R)

        wait(b, slot)
        reduce(b, slot)

    for r in range(_R - 1):
        issue(r, r)

    def outer(o, carry):
        b0 = o * _R
        for k in range(_R):
            step(b0 + k, k)
        return carry

    lax.fori_loop(0, _BPW // _R, outer, 0)

    for k in range(_BPW - _BPW % _R, _BPW):
        step(k, k % _R)

    pltpu.sync_copy(out_v, out_hbm.at[pl.ds(base, _BPW)])


def kernel(x, embed_weight):
    return _glove_bow_sc(x.astype(jnp.int32).reshape(-1), embed_weight)


# speedup vs baseline: 1.1497x; 1.0145x over previous
import functools

import jax
import jax.numpy as jnp
from jax import lax
from jax.experimental import pallas as pl
from jax.experimental.pallas import tpu as pltpu
from jax.experimental.pallas import tpu_sc as plsc

_B = 4096
_L = 200
_D = 64
_V = 1000000
_NC = 2
_NS = 16
_NW = _NC * _NS
_BPW = _B // _NW
_IPW = _BPW * _L
_RU = 8
_R = 4


@functools.partial(
    pl.kernel,
    mesh=plsc.VectorSubcoreMesh(core_axis_name="c", subcore_axis_name="s"),
    compiler_params=pltpu.CompilerParams(use_tc_tiling_on_sc=False),
    out_type=jax.ShapeDtypeStruct((_B, _D), jnp.float32),
    scratch_types=[
        pltpu.VMEM((_IPW,), jnp.int32),
        pltpu.VMEM((_R, _L, _D), jnp.float32),
        pltpu.VMEM((_BPW, _D), jnp.float32),
        pltpu.SemaphoreType.DMA,
        pltpu.SemaphoreType.DMA,
        pltpu.SemaphoreType.DMA,
        pltpu.SemaphoreType.DMA,
    ],
)
def _glove_bow_sc(x_hbm, tab_hbm, out_hbm, idx_v, rows_v, out_v,
                  sem0, sem1, sem2, sem3):
    wid = lax.axis_index("s") * _NC + lax.axis_index("c")
    base = wid * _BPW
    pltpu.sync_copy(x_hbm.at[pl.ds(base * _L, _IPW)], idx_v)

    sems = (sem0, sem1, sem2, sem3)

    def copies(b, slot):
        return (
            pltpu.make_async_copy(
                tab_hbm.at[idx_v.at[pl.ds(b * _L, _L)]],
                rows_v.at[slot],
                sems[slot],
            ),
        )

    def issue(b, slot):
        for cp in copies(b, slot):
            cp.start()

    def wait(b, slot):
        for cp in copies(b, slot):
            cp.wait()

    def reduce(b, slot):
        zero = jnp.zeros((16,), jnp.float32)

        def rbody(i, acc):
            l0 = i * _RU
            for u in range(_RU):
                l = l0 + u
                acc = tuple(
                    acc[j] + rows_v[slot, l, pl.ds(j * 16, 16)]
                    for j in range(4)
                )
            return acc

        acc = lax.fori_loop(0, _L // _RU, rbody, (zero,) * 4)
        for j in range(4):
            out_v[b, pl.ds(j * 16, 16)] = acc[j]

    def step(b, slot):
        nb = b + (_R - 1)

        @pl.when(nb < _BPW)
        def _():
            issue(nb, (slot + _R - 1) % _R)

        wait(b, slot)
        reduce(b, slot)

    for r in range(_R - 1):
        issue(r, r)

    def outer(o, carry):
        b0 = o * _R
        for k in range(_R):
            step(b0 + k, k)
        return carry

    lax.fori_loop(0, _BPW // _R, outer, 0)

    for k in range(_BPW - _BPW % _R, _BPW):
        step(k, k % _R)

    pltpu.sync_copy(out_v, out_hbm.at[pl.ds(base, _BPW)])


def kernel(x, embed_weight):
    return _glove_bow_sc(x.astype(jnp.int32).reshape(-1), embed_weight)
